# baseline (speedup 1.0000x reference)
import functools

import jax
import jax.numpy as jnp
from jax import lax
from jax.experimental import pallas as pl
from jax.experimental.pallas import tpu as pltpu
from jax.experimental.pallas import tpu_sc as plsc

N = 10000
NPAD = 10240
DIN = 128
D = 64
DT = 128
E = 320000
LANES = 128
CH = 80
NC = 2
NS = 16
NW = NC * NS
EPAD = NW * CH * LANES
RPS = NPAD // NS
BLK = 512
GRID = NPAD // BLK

_mesh = plsc.VectorSubcoreMesh(core_axis_name="c", subcore_axis_name="s")


NBUF = 8


def _seg_body(table_hbm, edges_hbm, out_hbm, src_idx, dst_idx, rows,
              acc, gsems, ssems):
    cid = lax.axis_index("c")
    sid = lax.axis_index("s")
    wid = sid * NC + cid

    pltpu.async_copy(edges_hbm.at[0, wid], src_idx, gsems[0])
    pltpu.async_copy(edges_hbm.at[1, wid], dst_idx, gsems[1])

    def fill(i, carry):
        for k in range(D // 16):
            rows[0][i, pl.ds(k * 16, 16)] = jnp.zeros((16,), jnp.float32)
        return carry

    lax.fori_loop(0, LANES, fill, 0)
    for k in range(RPS // LANES):
        pltpu.sync_copy(rows[0], acc.at[pl.ds(sid * RPS + k * LANES, LANES)])
    plsc.subcore_barrier()

    pltpu.make_async_copy(edges_hbm.at[0, wid], src_idx, gsems[0]).wait()
    pltpu.make_async_copy(edges_hbm.at[1, wid], dst_idx, gsems[1]).wait()

    for b in range(NBUF):
        pltpu.async_copy(table_hbm.at[src_idx.at[b]], rows[b], gsems[b])

    def step(i, carry):
        j = i * NBUF
        for b in range(NBUF):
            pltpu.make_async_copy(
                table_hbm.at[src_idx.at[j + b]], rows[b], gsems[b]).wait()
            pltpu.async_copy(rows[b], acc.at[dst_idx.at[j + b]], ssems[b],
                             add=True)
        for b in range(NBUF):
            pltpu.make_async_copy(
                rows[b], acc.at[dst_idx.at[j + b]], ssems[b]).wait()

            @pl.when(i + 1 < CH // NBUF)
            def _g():
                pltpu.async_copy(table_hbm.at[src_idx.at[j + NBUF + b]],
                                 rows[b], gsems[b])
        return carry

    lax.fori_loop(0, CH // NBUF, step, 0)
    plsc.subcore_barrier()

    pltpu.sync_copy(acc.at[pl.ds(sid * RPS, RPS)],
                    out_hbm.at[cid, pl.ds(sid * RPS, RPS)])


_seg_kernel = pl.kernel(
    _seg_body,
    mesh=_mesh,
    out_type=jax.ShapeDtypeStruct((NC, NPAD, D), jnp.float32),
    scratch_types=[
        pltpu.VMEM((CH, LANES), jnp.int32),
        pltpu.VMEM((CH, LANES), jnp.int32),
        [pltpu.VMEM((LANES, D), jnp.float32) for _ in range(NBUF)],
        pltpu.VMEM_SHARED((NPAD, D), jnp.float32),
        [pltpu.SemaphoreType.DMA for _ in range(NBUF)],
        [pltpu.SemaphoreType.DMA for _ in range(NBUF)],
    ],
    compiler_params=pltpu.CompilerParams(use_tc_tiling_on_sc=False),
)


def _deg_body2(edges_hbm, out_hbm, dst_idx, ones_v, zbuf, acc, dsems):
    cid = lax.axis_index("c")
    sid = lax.axis_index("s")
    wid = sid * NC + cid

    def fill(i, carry):
        zbuf[pl.ds(i * 16, 16)] = jnp.zeros((16,), jnp.float32)
        return carry

    lax.fori_loop(0, RPS // 16, fill, 0)
    for k in range(LANES // 16):
        ones_v[pl.ds(k * 16, 16)] = jnp.ones((16,), jnp.float32)

    pltpu.sync_copy(zbuf, acc.at[pl.ds(sid * RPS, RPS)])
    plsc.subcore_barrier()

    pltpu.sync_copy(edges_hbm.at[1, wid], dst_idx)

    def step(i, carry):
        j = i * NBUF
        for b in range(NBUF):
            @pl.when(i > 0)
            def _w():
                pltpu.make_async_copy(
                    ones_v, acc.at[dst_idx.at[j - NBUF + b]], dsems[b]).wait()

            pltpu.async_copy(ones_v, acc.at[dst_idx.at[j + b]], dsems[b],
                             add=True)
        return carry

    lax.fori_loop(0, CH // NBUF, step, 0)
    for b in range(NBUF):
        pltpu.make_async_copy(
            ones_v, acc.at[dst_idx.at[CH - NBUF + b]], dsems[b]).wait()
    plsc.subcore_barrier()

    pltpu.sync_copy(acc.at[pl.ds(sid * RPS, RPS)],
                    out_hbm.at[cid, pl.ds(sid * RPS, RPS)])


_deg_kernel = pl.kernel(
    _deg_body2,
    mesh=_mesh,
    out_type=jax.ShapeDtypeStruct((NC, NPAD), jnp.float32),
    scratch_types=[
        pltpu.VMEM((CH, LANES), jnp.int32),
        pltpu.VMEM((LANES,), jnp.float32),
        pltpu.VMEM((RPS,), jnp.float32),
        pltpu.VMEM_SHARED((NPAD,), jnp.float32),
        [pltpu.SemaphoreType.DMA for _ in range(NBUF)],
    ],
    compiler_params=pltpu.CompilerParams(use_tc_tiling_on_sc=False),
)


def _mm_body(x_ref, w_ref, o_ref):
    o_ref[...] = lax.dot_general(
        x_ref[...], w_ref[...], (((1,), (0,)), ((), ())),
        preferred_element_type=jnp.float32,
        precision=lax.Precision.HIGHEST)


_mm = pl.pallas_call(
    _mm_body,
    grid=(GRID,),
    in_specs=[
        pl.BlockSpec((BLK, DIN), lambda i: (i, 0)),
        pl.BlockSpec((DIN, D), lambda i: (0, 0)),
    ],
    out_specs=pl.BlockSpec((BLK, D), lambda i: (i, 0)),
    out_shape=jax.ShapeDtypeStruct((NPAD, D), jnp.float32),
)


def _scale_body(deg_ref, xw_ref, t_ref, dinv_ref):
    degt = deg_ref[0] + deg_ref[1] + 1.0
    dinv = lax.rsqrt(degt)
    dinv_ref[...] = dinv
    t_ref[...] = xw_ref[...] * dinv


_scale = pl.pallas_call(
    _scale_body,
    grid=(GRID,),
    in_specs=[
        pl.BlockSpec((NC, BLK, 1), lambda i: (0, i, 0)),
        pl.BlockSpec((BLK, D), lambda i: (i, 0)),
    ],
    out_specs=[
        pl.BlockSpec((BLK, D), lambda i: (i, 0)),
        pl.BlockSpec((BLK, 1), lambda i: (i, 0)),
    ],
    out_shape=[
        jax.ShapeDtypeStruct((NPAD, D), jnp.float32),
        jax.ShapeDtypeStruct((NPAD, 1), jnp.float32),
    ],
)


def _layer_body(t1_ref, seg_ref, dinv_ref, b_ref, w_ref, t2_ref):
    dinv = dinv_ref[...]
    seg = seg_ref[0] + seg_ref[1]
    pre = (seg + t1_ref[...]) * dinv + b_ref[...][None, :]
    h = jnp.maximum(pre, 0.0)
    xw2 = lax.dot_general(
        h, w_ref[...], (((1,), (0,)), ((), ())),
        preferred_element_type=jnp.float32,
        precision=lax.Precision.HIGHEST)
    t2_ref[...] = xw2 * dinv


_layer = pl.pallas_call(
    _layer_body,
    grid=(GRID,),
    in_specs=[
        pl.BlockSpec((BLK, D), lambda i: (i, 0)),
        pl.BlockSpec((NC, BLK, D), lambda i: (0, i, 0)),
        pl.BlockSpec((BLK, 1), lambda i: (i, 0)),
        pl.BlockSpec((D,), lambda i: (0,)),
        pl.BlockSpec((D, D), lambda i: (0, 0)),
    ],
    out_specs=pl.BlockSpec((BLK, D), lambda i: (i, 0)),
    out_shape=jax.ShapeDtypeStruct((NPAD, D), jnp.float32),
)


def _final_body(t2_ref, seg_ref, dinv_ref, b_ref, o_ref):
    i = pl.program_id(0)
    seg = seg_ref[0] + seg_ref[1]
    pre = (seg + t2_ref[...]) * dinv_ref[...] + b_ref[...][None, :]
    h = jnp.maximum(pre, 0.0)
    row = lax.broadcasted_iota(jnp.int32, (BLK, 1), 0) + i * BLK
    h = jnp.where(row < N, h, 0.0)
    s = jnp.sum(h, axis=0) * (1.0 / N)

    @pl.when(i == 0)
    def _init():
        o_ref[...] = s

    @pl.when(i > 0)
    def _acc():
        o_ref[...] = o_ref[...] + s


_final = pl.pallas_call(
    _final_body,
    grid=(GRID,),
    in_specs=[
        pl.BlockSpec((BLK, D), lambda i: (i, 0)),
        pl.BlockSpec((NC, BLK, D), lambda i: (0, i, 0)),
        pl.BlockSpec((BLK, 1), lambda i: (i, 0)),
        pl.BlockSpec((D,), lambda i: (0,)),
    ],
    out_specs=pl.BlockSpec((D,), lambda i: (0,)),
    out_shape=jax.ShapeDtypeStruct((D,), jnp.float32),
)


def kernel(x, edge_index, W1, b1, W2, b2):
    pad = (jnp.arange(EPAD - E, dtype=jnp.int32) % (NPAD - N)) + N
    edges = jnp.concatenate(
        [edge_index, jnp.broadcast_to(pad, (2, EPAD - E))], axis=1
    ).reshape(2, NW, CH, LANES)
    xp = jnp.pad(x, ((0, NPAD - N), (0, 0)))

    deg = _deg_kernel(edges)
    xw1 = _mm(xp, W1)
    t1, dinv = _scale(jnp.reshape(deg, (NC, NPAD, 1)), xw1)
    seg1 = _seg_kernel(t1, edges)
    t2 = _layer(t1, seg1, dinv, b1, W2)
    seg2 = _seg_kernel(t2, edges)
    return _final(t2, seg2, dinv, b2)


# speedup vs baseline: 1.0512x; 1.0512x over previous
import functools

import jax
import jax.numpy as jnp
from jax import lax
from jax.experimental import pallas as pl
from jax.experimental.pallas import tpu as pltpu
from jax.experimental.pallas import tpu_sc as plsc

N = 10000
NPAD = 10240
DIN = 128
D = 64
DT = 128
E = 320000
LANES = 128
CH = 80
NC = 2
NS = 16
NW = NC * NS
EPAD = NW * CH * LANES
RPS = NPAD // NS
BLK = 512
GRID = NPAD // BLK

_mesh = plsc.VectorSubcoreMesh(core_axis_name="c", subcore_axis_name="s")


NBUF = 8


def _seg_body(table_hbm, edges_hbm, out_hbm, src_idx, dst_idx, rows,
              acc, gsems, ssems):
    cid = lax.axis_index("c")
    sid = lax.axis_index("s")
    wid = sid * NC + cid

    pltpu.async_copy(edges_hbm.at[0, wid], src_idx, gsems[0])
    pltpu.async_copy(edges_hbm.at[1, wid], dst_idx, gsems[1])

    def fill(i, carry):
        for k in range(D // 16):
            rows[0][i, pl.ds(k * 16, 16)] = jnp.zeros((16,), jnp.float32)
        return carry

    lax.fori_loop(0, LANES, fill, 0)
    for k in range(RPS // LANES):
        pltpu.sync_copy(rows[0], acc.at[pl.ds(sid * RPS + k * LANES, LANES)])
    plsc.subcore_barrier()

    pltpu.make_async_copy(edges_hbm.at[0, wid], src_idx, gsems[0]).wait()
    pltpu.make_async_copy(edges_hbm.at[1, wid], dst_idx, gsems[1]).wait()

    for b in range(NBUF):
        pltpu.async_copy(table_hbm.at[src_idx.at[b]], rows[b], gsems[b])

    def step(i, carry):
        j = i * NBUF
        for b in range(NBUF):
            pltpu.make_async_copy(
                table_hbm.at[src_idx.at[j + b]], rows[b], gsems[b]).wait()
            pltpu.async_copy(rows[b], acc.at[dst_idx.at[j + b]], ssems[b],
                             add=True)
        for b in range(NBUF):
            pltpu.make_async_copy(
                rows[b], acc.at[dst_idx.at[j + b]], ssems[b]).wait()

            @pl.when(i + 1 < CH // NBUF)
            def _g():
                pltpu.async_copy(table_hbm.at[src_idx.at[j + NBUF + b]],
                                 rows[b], gsems[b])
        return carry

    lax.fori_loop(0, CH // NBUF, step, 0)
    plsc.subcore_barrier()

    pltpu.sync_copy(acc.at[pl.ds(sid * RPS, RPS)],
                    out_hbm.at[cid, pl.ds(sid * RPS, RPS)])


_seg_kernel = pl.kernel(
    _seg_body,
    mesh=_mesh,
    out_type=jax.ShapeDtypeStruct((NC, NPAD, D), jnp.float32),
    scratch_types=[
        pltpu.VMEM((CH, LANES), jnp.int32),
        pltpu.VMEM((CH, LANES), jnp.int32),
        [pltpu.VMEM((LANES, D), jnp.float32) for _ in range(NBUF)],
        pltpu.VMEM_SHARED((NPAD, D), jnp.float32),
        [pltpu.SemaphoreType.DMA for _ in range(NBUF)],
        [pltpu.SemaphoreType.DMA for _ in range(NBUF)],
    ],
    compiler_params=pltpu.CompilerParams(use_tc_tiling_on_sc=False),
)


def _deg_body2(edges_hbm, out_hbm, dst_idx, ones_v, zbuf, acc, dsems):
    cid = lax.axis_index("c")
    sid = lax.axis_index("s")
    wid = sid * NC + cid

    def fill(i, carry):
        zbuf[pl.ds(i * 16, 16)] = jnp.zeros((16,), jnp.float32)
        return carry

    lax.fori_loop(0, RPS // 16, fill, 0)
    for k in range(LANES // 16):
        ones_v[pl.ds(k * 16, 16)] = jnp.ones((16,), jnp.float32)

    pltpu.sync_copy(zbuf, acc.at[pl.ds(sid * RPS, RPS)])
    plsc.subcore_barrier()

    pltpu.sync_copy(edges_hbm.at[1, wid], dst_idx)

    def step(i, carry):
        j = i * NBUF
        for b in range(NBUF):
            @pl.when(i > 0)
            def _w():
                pltpu.make_async_copy(
                    ones_v, acc.at[dst_idx.at[j - NBUF + b]], dsems[b]).wait()

            pltpu.async_copy(ones_v, acc.at[dst_idx.at[j + b]], dsems[b],
                             add=True)
        return carry

    lax.fori_loop(0, CH // NBUF, step, 0)
    for b in range(NBUF):
        pltpu.make_async_copy(
            ones_v, acc.at[dst_idx.at[CH - NBUF + b]], dsems[b]).wait()
    plsc.subcore_barrier()

    pltpu.sync_copy(acc.at[pl.ds(sid * RPS, RPS)],
                    out_hbm.at[cid, pl.ds(sid * RPS, RPS)])


_deg_kernel = pl.kernel(
    _deg_body2,
    mesh=_mesh,
    out_type=jax.ShapeDtypeStruct((NC, NPAD), jnp.float32),
    scratch_types=[
        pltpu.VMEM((CH, LANES), jnp.int32),
        pltpu.VMEM((LANES,), jnp.float32),
        pltpu.VMEM((RPS,), jnp.float32),
        pltpu.VMEM_SHARED((NPAD,), jnp.float32),
        [pltpu.SemaphoreType.DMA for _ in range(NBUF)],
    ],
    compiler_params=pltpu.CompilerParams(use_tc_tiling_on_sc=False),
)


def _scale_body(deg_ref, x_ref, w_ref, t_ref, dinv_ref):
    degt = deg_ref[0] + deg_ref[1] + 1.0
    dinv = lax.rsqrt(degt)
    xw = lax.dot_general(
        x_ref[...], w_ref[...], (((1,), (0,)), ((), ())),
        preferred_element_type=jnp.float32,
        precision=lax.Precision.HIGHEST)
    dinv_ref[...] = dinv
    t_ref[...] = xw * dinv


_scale = pl.pallas_call(
    _scale_body,
    grid=(GRID,),
    in_specs=[
        pl.BlockSpec((NC, BLK, 1), lambda i: (0, i, 0)),
        pl.BlockSpec((BLK, DIN), lambda i: (i, 0)),
        pl.BlockSpec((DIN, D), lambda i: (0, 0)),
    ],
    out_specs=[
        pl.BlockSpec((BLK, D), lambda i: (i, 0)),
        pl.BlockSpec((BLK, 1), lambda i: (i, 0)),
    ],
    out_shape=[
        jax.ShapeDtypeStruct((NPAD, D), jnp.float32),
        jax.ShapeDtypeStruct((NPAD, 1), jnp.float32),
    ],
)


def _layer_body(t1_ref, seg_ref, dinv_ref, b_ref, w_ref, t2_ref):
    dinv = dinv_ref[...]
    seg = seg_ref[0] + seg_ref[1]
    pre = (seg + t1_ref[...]) * dinv + b_ref[...][None, :]
    h = jnp.maximum(pre, 0.0)
    xw2 = lax.dot_general(
        h, w_ref[...], (((1,), (0,)), ((), ())),
        preferred_element_type=jnp.float32,
        precision=lax.Precision.HIGHEST)
    t2_ref[...] = xw2 * dinv


_layer = pl.pallas_call(
    _layer_body,
    grid=(GRID,),
    in_specs=[
        pl.BlockSpec((BLK, D), lambda i: (i, 0)),
        pl.BlockSpec((NC, BLK, D), lambda i: (0, i, 0)),
        pl.BlockSpec((BLK, 1), lambda i: (i, 0)),
        pl.BlockSpec((D,), lambda i: (0,)),
        pl.BlockSpec((D, D), lambda i: (0, 0)),
    ],
    out_specs=pl.BlockSpec((BLK, D), lambda i: (i, 0)),
    out_shape=jax.ShapeDtypeStruct((NPAD, D), jnp.float32),
)


def _final_body(t2_ref, seg_ref, dinv_ref, b_ref, o_ref):
    i = pl.program_id(0)
    seg = seg_ref[0] + seg_ref[1]
    pre = (seg + t2_ref[...]) * dinv_ref[...] + b_ref[...][None, :]
    h = jnp.maximum(pre, 0.0)
    row = lax.broadcasted_iota(jnp.int32, (BLK, 1), 0) + i * BLK
    h = jnp.where(row < N, h, 0.0)
    s = jnp.sum(h, axis=0) * (1.0 / N)

    @pl.when(i == 0)
    def _init():
        o_ref[...] = s

    @pl.when(i > 0)
    def _acc():
        o_ref[...] = o_ref[...] + s


_final = pl.pallas_call(
    _final_body,
    grid=(GRID,),
    in_specs=[
        pl.BlockSpec((BLK, D), lambda i: (i, 0)),
        pl.BlockSpec((NC, BLK, D), lambda i: (0, i, 0)),
        pl.BlockSpec((BLK, 1), lambda i: (i, 0)),
        pl.BlockSpec((D,), lambda i: (0,)),
    ],
    out_specs=pl.BlockSpec((D,), lambda i: (0,)),
    out_shape=jax.ShapeDtypeStruct((D,), jnp.float32),
)


def kernel(x, edge_index, W1, b1, W2, b2):
    pad = (jnp.arange(EPAD - E, dtype=jnp.int32) % (NPAD - N)) + N
    edges = jnp.concatenate(
        [edge_index, jnp.broadcast_to(pad, (2, EPAD - E))], axis=1
    ).reshape(2, NW, CH, LANES)
    xp = jnp.pad(x, ((0, NPAD - N), (0, 0)))

    deg = _deg_kernel(edges)
    t1, dinv = _scale(jnp.reshape(deg, (NC, NPAD, 1)), xp, W1)
    seg1 = _seg_kernel(t1, edges)
    t2 = _layer(t1, seg1, dinv, b1, W2)
    seg2 = _seg_kernel(t2, edges)
    return _final(t2, seg2, dinv, b2)
